# baseline (speedup 1.0000x reference)
import functools

import jax
import jax.numpy as jnp
from jax import lax
from jax.experimental import pallas as pl
from jax.experimental.pallas import tpu as pltpu
from jax.experimental.pallas import tpu_sc as plsc

_RES = 1000000
_LAT = 64
_N = 1048576
_NC = 2
_NS = 16
_NW = _NC * _NS
_BW = _N // _NW
_C = 128
_G = _BW // _C


def _body(u_hbm, table_hbm, out_hbm,
          u_full, idx_a0, idx_a1, idx_b0, idx_b1,
          rows_a0, rows_a1, rows_b0, rows_b1, rows_o0, rows_o1,
          sem_g0, sem_g1, sem_o0, sem_o1):
    wid = lax.axis_index("s") * _NC + lax.axis_index("c")
    base0 = wid * _BW
    scale = jnp.float32(_RES - 1)
    idx_a = (idx_a0, idx_a1)
    idx_b = (idx_b0, idx_b1)
    rows_a = (rows_a0, rows_a1)
    rows_b = (rows_b0, rows_b1)
    rows_o = (rows_o0, rows_o1)
    sem_g = (sem_g0, sem_g1)
    sem_o = (sem_o0, sem_o1)
    lane = lax.iota(jnp.int32, 16)

    def gather_copies(b):
        return [
            pltpu.make_async_copy(table_hbm.at[idx_a[b]], rows_a[b], sem_g[b]),
            pltpu.make_async_copy(table_hbm.at[idx_b[b]], rows_b[b], sem_g[b]),
        ]

    def out_copy(g, b):
        off = pl.multiple_of(wid * (_BW // 2) + g * (_C // 2), _C // 2)
        return pltpu.make_async_copy(
            rows_o[b], out_hbm.at[pl.ds(off, _C // 2)], sem_o[b])

    def prep(g, b):
        def idx_body(k, c):
            off = k * 16 + lane
            goff = g * _C + off
            u16 = plsc.load_gather(u_full, [goff])
            f = u16 * scale
            ix = f.astype(jnp.int32)
            fl = ix.astype(jnp.float32)
            plsc.store_scatter(idx_a[b], [off], lax.shift_right_logical(ix, 1))
            plsc.store_scatter(idx_b[b], [off],
                               lax.shift_right_logical(ix + 1, 1))
            plsc.store_scatter(u_full, [goff], f - fl)
            return c

        lax.fori_loop(0, _C // 16, idx_body, 0, unroll=True)
        for c in gather_copies(b):
            c.start()

    def cons(g, b, first):
        for c in gather_copies(b):
            c.wait()
        if not first:
            out_copy(g, b).wait()

        def lerp_body(blk, c):
            li = blk * 16 + lane
            al = plsc.load_gather(u_full, [g * _C + li])
            ia = plsc.load_gather(idx_a[b], [li])
            ib = plsc.load_gather(idx_b[b], [li])
            par = ib - ia
            acol0 = par * 64
            bcol0 = (1 - par) * 64
            orow = lax.shift_right_logical(li, 1)
            ocol0 = (li & 1) * 64
            for col in range(_LAT):
                av = plsc.load_gather(rows_a[b], [li, acol0 + col])
                bv = plsc.load_gather(rows_b[b], [li, bcol0 + col])
                ov = av + al * (bv - av)
                plsc.store_scatter(rows_o[b], [orow, ocol0 + col], ov)
            return c

        lax.fori_loop(0, _C // 16, lerp_body, 0, unroll=False)
        out_copy(g, b).start()

    pltpu.sync_copy(u_hbm.at[pl.ds(pl.multiple_of(base0, _BW), _BW)], u_full)

    prep(0, 0)
    prep(1, 1)
    cons(0, 0, True)
    prep(2, 0)
    cons(1, 1, True)
    prep(3, 1)

    def pair(gg, carry):
        for b in range(2):
            g = gg * 2 + b
            cons(g, b, False)

            @pl.when(g + 2 < _G)
            def _():
                prep(g + 2, b)
        return carry

    lax.fori_loop(1, _G // 2, pair, 0, unroll=False)

    out_copy(_G - 2, 0).wait()
    out_copy(_G - 1, 1).wait()


def kernel(uList, table):
    mesh = plsc.VectorSubcoreMesh(core_axis_name="c", subcore_axis_name="s")
    table2 = table.reshape(_RES // 2, 2 * _LAT)
    k = functools.partial(
        pl.kernel,
        mesh=mesh,
        out_type=jax.ShapeDtypeStruct((_N // 2, 2 * _LAT), jnp.float32),
        compiler_params=pltpu.CompilerParams(needs_layout_passes=False),
        scratch_types=[
            pltpu.VMEM((_BW,), jnp.float32),
            pltpu.VMEM((_C,), jnp.int32),
            pltpu.VMEM((_C,), jnp.int32),
            pltpu.VMEM((_C,), jnp.int32),
            pltpu.VMEM((_C,), jnp.int32),
            pltpu.VMEM((_C, 2 * _LAT), jnp.float32),
            pltpu.VMEM((_C, 2 * _LAT), jnp.float32),
            pltpu.VMEM((_C, 2 * _LAT), jnp.float32),
            pltpu.VMEM((_C, 2 * _LAT), jnp.float32),
            pltpu.VMEM((_C // 2, 2 * _LAT), jnp.float32),
            pltpu.VMEM((_C // 2, 2 * _LAT), jnp.float32),
            pltpu.SemaphoreType.DMA,
            pltpu.SemaphoreType.DMA,
            pltpu.SemaphoreType.DMA,
            pltpu.SemaphoreType.DMA,
        ],
    )(_body)
    out2 = k(uList, table2)
    return out2.reshape(_N, _LAT)


# speedup vs baseline: 2.2416x; 2.2416x over previous
import functools

import jax
import jax.numpy as jnp
from jax import lax
from jax.experimental import pallas as pl
from jax.experimental.pallas import tpu as pltpu
from jax.experimental.pallas import tpu_sc as plsc

_RES = 1000000
_LAT = 64
_N = 1048576
_NC = 2
_NS = 16
_NW = _NC * _NS
_BW = _N // _NW
_C = 128
_G = _BW // _C
_NSLOT = 3


def _body(u_hbm, table_hbm, out_hbm, *args):
    u_v = args[0:3]
    idx_a = args[3:6]
    idx_b = args[6:9]
    rows_a = args[9:12]
    rows_b = args[12:15]
    rows_o = args[15:18]
    sem_g = args[18:21]
    sem_o = args[21:24]
    wid = lax.axis_index("s") * _NC + lax.axis_index("c")
    base0 = wid * _BW
    scale = jnp.float32(_RES - 1)

    def gather_copies(b):
        return [
            pltpu.make_async_copy(
                table_hbm.at[idx_a[b].at[0]], rows_a[b], sem_g[b]),
            pltpu.make_async_copy(
                table_hbm.at[idx_b[b].at[0]], rows_b[b], sem_g[b]),
        ]

    def out_copy(g, b):
        return pltpu.make_async_copy(
            rows_o[b],
            out_hbm.at[pl.ds(wid * (_BW // 2) + g * (_C // 2), _C // 2)],
            sem_o[b])

    def prep(g, b):
        base = base0 + g * _C
        pltpu.sync_copy(u_hbm.at[pl.ds(base, _C)], u_v[b])

        def idx_body(k, c):
            off = k * 16
            u16 = u_v[b][pl.ds(off, 16)]
            f = u16 * scale
            ix = f.astype(jnp.int32)
            fl = ix.astype(jnp.float32)
            idx_a[b][0, pl.ds(off, 16)] = lax.shift_right_logical(ix, 1)
            idx_b[b][0, pl.ds(off, 16)] = lax.shift_right_logical(ix + 1, 1)
            u_v[b][pl.ds(off, 16)] = f - fl
            return c

        lax.fori_loop(0, _C // 16, idx_body, 0, unroll=True)
        for c in gather_copies(b):
            c.start()

    def cons(g, b, first):
        for c in gather_copies(b):
            c.wait()
        if not first:
            out_copy(g, b).wait()

        def lerp_body(blk, c):
            i0 = blk * 16
            al16 = u_v[b][pl.ds(i0, 16)]
            ia16 = idx_a[b][0, pl.ds(i0, 16)]
            ib16 = idx_b[b][0, pl.ds(i0, 16)]
            par16 = ib16 - ia16
            for l in range(16):
                i = i0 + l
                al = jnp.full((16,), al16[l], jnp.float32)
                acol = par16[l] * 64
                bcol = 64 - acol
                orow = blk * 8 + (l // 2)
                ocol = (l & 1) * 64
                for c4 in range(4):
                    a = rows_a[b][i, pl.ds(acol + c4 * 16, 16)]
                    bb = rows_b[b][i, pl.ds(bcol + c4 * 16, 16)]
                    rows_o[b][orow, pl.ds(ocol + c4 * 16, 16)] = (
                        a + al * (bb - a))
            return c

        lax.fori_loop(0, _C // 16, lerp_body, 0, unroll=False)
        out_copy(g, b).start()

    prep(0, 0)
    prep(1, 1)
    prep(2, 2)
    cons(0, 0, True)
    prep(3, 0)
    cons(1, 1, True)
    prep(4, 1)
    cons(2, 2, True)
    prep(5, 2)
    cons(3, 0, False)
    prep(6, 0)

    def triple(t, carry):
        for j in range(3):
            g = 4 + t * 3 + j
            b = (4 + j) % _NSLOT
            cons(g, b, False)

            @pl.when(g + 3 < _G)
            def _():
                prep(g + 3, b)
        return carry

    lax.fori_loop(0, (_G - 4) // 3, triple, 0, unroll=False)

    out_copy(_G - 3, (_G - 3) % _NSLOT).wait()
    out_copy(_G - 2, (_G - 2) % _NSLOT).wait()
    out_copy(_G - 1, (_G - 1) % _NSLOT).wait()


def kernel(uList, table):
    mesh = plsc.VectorSubcoreMesh(core_axis_name="c", subcore_axis_name="s")
    table2 = table.reshape(_RES // 2, 2 * _LAT)
    scr = (
        [pltpu.VMEM((_C,), jnp.float32) for _ in range(_NSLOT)]
        + [pltpu.VMEM((1, _C), jnp.int32) for _ in range(_NSLOT)]
        + [pltpu.VMEM((1, _C), jnp.int32) for _ in range(_NSLOT)]
        + [pltpu.VMEM((_C, 2 * _LAT), jnp.float32) for _ in range(_NSLOT)]
        + [pltpu.VMEM((_C, 2 * _LAT), jnp.float32) for _ in range(_NSLOT)]
        + [pltpu.VMEM((_C // 2, 2 * _LAT), jnp.float32) for _ in range(_NSLOT)]
        + [pltpu.SemaphoreType.DMA for _ in range(2 * _NSLOT)]
    )
    k = functools.partial(
        pl.kernel,
        mesh=mesh,
        out_type=jax.ShapeDtypeStruct((_N // 2, 2 * _LAT), jnp.float32),
        compiler_params=pltpu.CompilerParams(use_tc_tiling_on_sc=False),
        scratch_types=scr,
    )(_body)
    out2 = k(uList, table2)
    return out2.reshape(_N, _LAT)
